# Initial kernel scaffold; baseline (speedup 1.0000x reference)
import jax
import jax.numpy as jnp
from jax.experimental import pallas as pl


def kernel(states, position, velocity, heading, predict_mask, agent_index, W_enc, W_dec_pos, W_dec_head, W1, b1, W2, b2, W3, b3):
    raise NotImplementedError("write your pallas kernel here")


# speedup vs baseline: 1.6403x; 1.6403x over previous
import functools

import jax
import jax.numpy as jnp
from jax.experimental import pallas as pl
from jax.experimental.pallas import tpu as pltpu

N_AGENTS = 10000
T_HIST = 50
T_FUT = 60
T_TOT = T_HIST + T_FUT
D_ENC = 512
HIDDEN = 4096
HALF = HIDDEN // 2
CHUNK = 512
GRID = HIDDEN // CHUNK


def _policy_kernel(idx_ref, states_ref, pos_ref, vel_ref, head_ref,
                   wxT_ref, wyT_ref, wdpxT_ref, wdpyT_ref, wdhT_ref,
                   w1_ref, b1_ref, w2_ref, b2_ref, w3_ref, b3_ref,
                   mean_ref, std_ref, act_s, acc_s):
    g = pl.program_id(0)

    @pl.when(g == 0)
    def _build_action():
        x_col = pos_ref[0, :, 0:1]
        y_col = pos_ref[0, :, 1:2]
        st = states_ref[...]
        st_pad = jnp.pad(st, ((T_HIST, 0), (0, 0)))
        t110 = jax.lax.broadcasted_iota(jnp.int32, (T_TOT, 1), 0)
        fut = t110 >= T_HIST
        x_col = jnp.where(fut, st_pad[:, 0:1], x_col)
        y_col = jnp.where(fut, st_pad[:, 1:2], y_col)
        feat = jnp.tanh(
            jnp.dot(wxT_ref[...], x_col, preferred_element_type=jnp.float32)
            + jnp.dot(wyT_ref[...], y_col, preferred_element_type=jnp.float32))
        lpx = jnp.dot(wdpxT_ref[...], feat, preferred_element_type=jnp.float32)
        lpy = jnp.dot(wdpyT_ref[...], feat, preferred_element_type=jnp.float32)
        lh = jnp.dot(wdhT_ref[...], feat, preferred_element_type=jnp.float32)
        theta = head_ref[0, 0, T_HIST - 1]
        c, s = jnp.cos(theta), jnp.sin(theta)
        ox = pos_ref[0, T_HIST - 1, 0]
        oy = pos_ref[0, T_HIST - 1, 1]
        npx = lpx * c - lpy * s + ox
        npy = lpx * s + lpy * c + oy
        t60 = jax.lax.broadcasted_iota(jnp.int32, (T_FUT, 1), 0)
        z1 = jnp.zeros((1, 1), jnp.float32)
        px_prev = jnp.concatenate([z1, npx[:-1, :]], axis=0)
        py_prev = jnp.concatenate([z1, npy[:-1, :]], axis=0)
        vx = jnp.where(t60 == 0, npx - ox, (npx - px_prev) * 10.0)
        vy = jnp.where(t60 == 0, npy - oy, (npy - py_prev) * 10.0)
        v49x = vel_ref[0, T_HIST - 1, 0]
        v49y = vel_ref[0, T_HIST - 1, 1]
        vx_prev = jnp.concatenate([z1, vx[:-1, :]], axis=0)
        vy_prev = jnp.concatenate([z1, vy[:-1, :]], axis=0)
        ax = jnp.where(t60 == 0, vx - v49x, (vx - vx_prev) * 10.0)
        ay = jnp.where(t60 == 0, vy - v49y, (vy - vy_prev) * 10.0)
        two_pi = 2.0 * jnp.pi
        hd = lh + theta
        hd = (hd + jnp.pi) % two_pi - jnp.pi
        act_s[...] = jnp.concatenate([hd, ax, ay], axis=1)
        acc_s[...] = jnp.zeros_like(acc_s)

    hd = act_s[:, 0:1]
    ax = act_s[:, 1:2]
    ay = act_s[:, 2:3]
    h1 = (hd * w1_ref[0:1, :] + ax * w1_ref[1:2, :] + ay * w1_ref[2:3, :]
          + b1_ref[...])
    h1 = jnp.maximum(h1, 0.0)
    acc_s[...] += jnp.dot(h1, w2_ref[...], preferred_element_type=jnp.float32)

    @pl.when(g == GRID - 1)
    def _finalize():
        h2 = acc_s[...] + b2_ref[...]
        y = jnp.dot(h2, w3_ref[...], preferred_element_type=jnp.float32) \
            + b3_ref[...]
        mean_ref[...] = jnp.tanh(y)
        std_ref[...] = jnp.log1p(jnp.exp(-jnp.abs(y))) + jnp.maximum(y, 0.0) \
            + 1e-8


@functools.partial(jax.jit, static_argnames=())
def kernel(states, position, velocity, heading, predict_mask, agent_index,
           W_enc, W_dec_pos, W_dec_head, W1, b1, W2, b2, W3, b3):
    del predict_mask
    idx = jnp.asarray(agent_index, jnp.int32).reshape((1,))
    wxT = W_enc[0::2, :].T
    wyT = W_enc[1::2, :].T
    wdpxT = W_dec_pos[:, 0:2 * T_FUT:2].T
    wdpyT = W_dec_pos[:, 1:2 * T_FUT:2].T
    wdhT = W_dec_head[:, 0:T_FUT].T
    heading3 = heading.reshape(N_AGENTS, 1, T_TOT)
    b1r = b1.reshape(1, HIDDEN)
    b2r = b2.reshape(1, HALF)
    b3r = b3.reshape(1, 3)

    grid_spec = pltpu.PrefetchScalarGridSpec(
        num_scalar_prefetch=1,
        grid=(GRID,),
        in_specs=[
            pl.BlockSpec((T_FUT, 4), lambda g, i: (0, 0)),
            pl.BlockSpec((1, T_TOT, 3), lambda g, i: (i[0], 0, 0)),
            pl.BlockSpec((1, T_TOT, 3), lambda g, i: (i[0], 0, 0)),
            pl.BlockSpec((1, 1, T_TOT), lambda g, i: (i[0], 0, 0)),
            pl.BlockSpec((D_ENC, T_TOT), lambda g, i: (0, 0)),
            pl.BlockSpec((D_ENC, T_TOT), lambda g, i: (0, 0)),
            pl.BlockSpec((T_FUT, D_ENC), lambda g, i: (0, 0)),
            pl.BlockSpec((T_FUT, D_ENC), lambda g, i: (0, 0)),
            pl.BlockSpec((T_FUT, D_ENC), lambda g, i: (0, 0)),
            pl.BlockSpec((3, CHUNK), lambda g, i: (0, g)),
            pl.BlockSpec((1, CHUNK), lambda g, i: (0, g)),
            pl.BlockSpec((CHUNK, HALF), lambda g, i: (g, 0)),
            pl.BlockSpec((1, HALF), lambda g, i: (0, 0)),
            pl.BlockSpec((HALF, 3), lambda g, i: (0, 0)),
            pl.BlockSpec((1, 3), lambda g, i: (0, 0)),
        ],
        out_specs=[
            pl.BlockSpec((T_FUT, 3), lambda g, i: (0, 0)),
            pl.BlockSpec((T_FUT, 3), lambda g, i: (0, 0)),
        ],
        scratch_shapes=[
            pltpu.VMEM((T_FUT, 3), jnp.float32),
            pltpu.VMEM((T_FUT, HALF), jnp.float32),
        ],
    )
    mean, std = pl.pallas_call(
        _policy_kernel,
        grid_spec=grid_spec,
        out_shape=[
            jax.ShapeDtypeStruct((T_FUT, 3), jnp.float32),
            jax.ShapeDtypeStruct((T_FUT, 3), jnp.float32),
        ],
    )(idx, states, position, velocity, heading3,
      wxT, wyT, wdpxT, wdpyT, wdhT,
      W1, b1r, W2, b2r, W3, b3r)
    return (mean, std)


# speedup vs baseline: 27.0560x; 16.4942x over previous
import jax
import jax.numpy as jnp
from jax.experimental import pallas as pl
from jax.experimental.pallas import tpu as pltpu

T_HIST = 50
T_FUT = 60
T_TOT = T_HIST + T_FUT
D_ENC = 512
HIDDEN = 4096
HALF = HIDDEN // 2
CHUNK = 512
GRID = HIDDEN // CHUNK


def _policy_kernel(states_ref, pos_ref, origin_ref,
                   wxT_ref, wyT_ref, wdpxT_ref, wdpyT_ref, wdhT_ref,
                   w1_ref, b1_ref, w2_ref, b2_ref, w3_ref, b3_ref,
                   mean_ref, std_ref, act_s, acc_s):
    g = pl.program_id(0)

    @pl.when(g == 0)
    def _build_action():
        x_col = pos_ref[:, 0:1]
        y_col = pos_ref[:, 1:2]
        st = states_ref[...]
        st_pad = jnp.pad(st, ((T_HIST, 0), (0, 0)))
        t110 = jax.lax.broadcasted_iota(jnp.int32, (T_TOT, 1), 0)
        fut = t110 >= T_HIST
        x_col = jnp.where(fut, st_pad[:, 0:1], x_col)
        y_col = jnp.where(fut, st_pad[:, 1:2], y_col)
        feat = jnp.tanh(
            jnp.dot(wxT_ref[...], x_col, preferred_element_type=jnp.float32)
            + jnp.dot(wyT_ref[...], y_col, preferred_element_type=jnp.float32))
        lpx = jnp.dot(wdpxT_ref[...], feat, preferred_element_type=jnp.float32)
        lpy = jnp.dot(wdpyT_ref[...], feat, preferred_element_type=jnp.float32)
        lh = jnp.dot(wdhT_ref[...], feat, preferred_element_type=jnp.float32)
        theta = origin_ref[0, 4]
        c, s = jnp.cos(theta), jnp.sin(theta)
        ox = origin_ref[0, 0]
        oy = origin_ref[0, 1]
        npx = lpx * c - lpy * s + ox
        npy = lpx * s + lpy * c + oy
        t60 = jax.lax.broadcasted_iota(jnp.int32, (T_FUT, 1), 0)
        z1 = jnp.zeros((1, 1), jnp.float32)
        px_prev = jnp.concatenate([z1, npx[:-1, :]], axis=0)
        py_prev = jnp.concatenate([z1, npy[:-1, :]], axis=0)
        vx = jnp.where(t60 == 0, npx - ox, (npx - px_prev) * 10.0)
        vy = jnp.where(t60 == 0, npy - oy, (npy - py_prev) * 10.0)
        v49x = origin_ref[0, 2]
        v49y = origin_ref[0, 3]
        vx_prev = jnp.concatenate([z1, vx[:-1, :]], axis=0)
        vy_prev = jnp.concatenate([z1, vy[:-1, :]], axis=0)
        ax = jnp.where(t60 == 0, vx - v49x, (vx - vx_prev) * 10.0)
        ay = jnp.where(t60 == 0, vy - v49y, (vy - vy_prev) * 10.0)
        two_pi = 2.0 * jnp.pi
        hd = lh + theta
        hd = (hd + jnp.pi) % two_pi - jnp.pi
        act_s[...] = jnp.concatenate([hd, ax, ay], axis=1)
        acc_s[...] = jnp.zeros_like(acc_s)

    hd = act_s[:, 0:1]
    ax = act_s[:, 1:2]
    ay = act_s[:, 2:3]
    h1 = (hd * w1_ref[0:1, :] + ax * w1_ref[1:2, :] + ay * w1_ref[2:3, :]
          + b1_ref[...])
    h1 = jnp.maximum(h1, 0.0)
    acc_s[...] += jnp.dot(h1, w2_ref[...], preferred_element_type=jnp.float32)

    @pl.when(g == GRID - 1)
    def _finalize():
        h2 = acc_s[...] + b2_ref[...]
        y = jnp.dot(h2, w3_ref[...], preferred_element_type=jnp.float32) \
            + b3_ref[...]
        mean_ref[...] = jnp.tanh(y)
        std_ref[...] = jnp.log1p(jnp.exp(-jnp.abs(y))) + jnp.maximum(y, 0.0) \
            + 1e-8


def kernel(states, position, velocity, heading, predict_mask, agent_index,
           W_enc, W_dec_pos, W_dec_head, W1, b1, W2, b2, W3, b3):
    del predict_mask
    idx = jnp.asarray(agent_index, jnp.int32)
    pos_row = jax.lax.dynamic_slice_in_dim(position, idx, 1, axis=0)
    pos_row = pos_row.reshape(T_TOT, 3)
    vel49 = jax.lax.dynamic_slice(velocity, (idx, T_HIST - 1, 0),
                                  (1, 1, 3)).reshape(1, 3)
    th49 = jax.lax.dynamic_slice(heading, (idx, T_HIST - 1), (1, 1))
    origin = jnp.concatenate(
        [pos_row[T_HIST - 1:T_HIST, 0:2], vel49[:, 0:2], th49], axis=1)
    wxT = W_enc[0::2, :].T
    wyT = W_enc[1::2, :].T
    wdpxT = W_dec_pos[:, 0:2 * T_FUT:2].T
    wdpyT = W_dec_pos[:, 1:2 * T_FUT:2].T
    wdhT = W_dec_head[:, 0:T_FUT].T
    b1r = b1.reshape(1, HIDDEN)
    b2r = b2.reshape(1, HALF)
    b3r = b3.reshape(1, 3)

    mean, std = pl.pallas_call(
        _policy_kernel,
        grid=(GRID,),
        in_specs=[
            pl.BlockSpec((T_FUT, 4), lambda g: (0, 0)),
            pl.BlockSpec((T_TOT, 3), lambda g: (0, 0)),
            pl.BlockSpec((1, 5), lambda g: (0, 0)),
            pl.BlockSpec((D_ENC, T_TOT), lambda g: (0, 0)),
            pl.BlockSpec((D_ENC, T_TOT), lambda g: (0, 0)),
            pl.BlockSpec((T_FUT, D_ENC), lambda g: (0, 0)),
            pl.BlockSpec((T_FUT, D_ENC), lambda g: (0, 0)),
            pl.BlockSpec((T_FUT, D_ENC), lambda g: (0, 0)),
            pl.BlockSpec((3, CHUNK), lambda g: (0, g)),
            pl.BlockSpec((1, CHUNK), lambda g: (0, g)),
            pl.BlockSpec((CHUNK, HALF), lambda g: (g, 0)),
            pl.BlockSpec((1, HALF), lambda g: (0, 0)),
            pl.BlockSpec((HALF, 3), lambda g: (0, 0)),
            pl.BlockSpec((1, 3), lambda g: (0, 0)),
        ],
        out_specs=[
            pl.BlockSpec((T_FUT, 3), lambda g: (0, 0)),
            pl.BlockSpec((T_FUT, 3), lambda g: (0, 0)),
        ],
        scratch_shapes=[
            pltpu.VMEM((T_FUT, 3), jnp.float32),
            pltpu.VMEM((T_FUT, HALF), jnp.float32),
        ],
        out_shape=[
            jax.ShapeDtypeStruct((T_FUT, 3), jnp.float32),
            jax.ShapeDtypeStruct((T_FUT, 3), jnp.float32),
        ],
    )(states, pos_row, origin,
      wxT, wyT, wdpxT, wdpyT, wdhT,
      W1, b1r, W2, b2r, W3, b3r)
    return (mean, std)


# speedup vs baseline: 29.1144x; 1.0761x over previous
import jax
import jax.numpy as jnp
from jax.experimental import pallas as pl
from jax.experimental.pallas import tpu as pltpu

T_HIST = 50
T_FUT = 60
T_TOT = T_HIST + T_FUT
TWO_T = 2 * T_TOT
D_ENC = 512
MODES6 = 6
HIDDEN = 4096
HALF = HIDDEN // 2
CHUNK = 512
GRID = HIDDEN // CHUNK

_TN = (((0,), (0,)), ((), ()))
_TT = (((0,), (1,)), ((), ()))


def _policy_kernel(states_ref, pos_ref, origin_ref,
                   wenc_ref, wdp_ref, wdh_ref,
                   w1_ref, b1_ref, w2_ref, b2_ref, w3_ref, b3_ref,
                   mean_ref, std_ref, act_s, acc_s):
    g = pl.program_id(0)

    @pl.when(g == 0)
    def _build_action():
        x_col = pos_ref[:, 0:1]
        y_col = pos_ref[:, 1:2]
        st = states_ref[...]
        st_pad = jnp.pad(st, ((T_HIST, 0), (0, 0)))
        t110 = jax.lax.broadcasted_iota(jnp.int32, (T_TOT, 1), 0)
        fut = t110 >= T_HIST
        x_col = jnp.where(fut, st_pad[:, 0:1], x_col)
        y_col = jnp.where(fut, st_pad[:, 1:2], y_col)
        r220 = jax.lax.broadcasted_iota(jnp.int32, (TWO_T, T_TOT), 0)
        c110 = jax.lax.broadcasted_iota(jnp.int32, (TWO_T, T_TOT), 1)
        px = (r220 == 2 * c110).astype(jnp.float32)
        py = (r220 == 2 * c110 + 1).astype(jnp.float32)
        xy = (jnp.dot(px, x_col, preferred_element_type=jnp.float32)
              + jnp.dot(py, y_col, preferred_element_type=jnp.float32))
        feat = jnp.tanh(jax.lax.dot_general(
            xy, wenc_ref[...], _TN, preferred_element_type=jnp.float32))
        dec = jnp.dot(feat, wdp_ref[...],
                      preferred_element_type=jnp.float32)
        dech = jnp.dot(feat, wdh_ref[...],
                       preferred_element_type=jnp.float32)
        rp = jax.lax.broadcasted_iota(jnp.int32, (MODES6 * T_FUT * 2, T_FUT), 0)
        cp = jax.lax.broadcasted_iota(jnp.int32, (MODES6 * T_FUT * 2, T_FUT), 1)
        sel_x = (rp == 2 * cp).astype(jnp.float32)
        sel_y = (rp == 2 * cp + 1).astype(jnp.float32)
        rh = jax.lax.broadcasted_iota(jnp.int32, (MODES6 * T_FUT, T_FUT), 0)
        ch = jax.lax.broadcasted_iota(jnp.int32, (MODES6 * T_FUT, T_FUT), 1)
        sel_h = (rh == ch).astype(jnp.float32)
        lpx = jax.lax.dot_general(sel_x, dec, _TT,
                                  preferred_element_type=jnp.float32)
        lpy = jax.lax.dot_general(sel_y, dec, _TT,
                                  preferred_element_type=jnp.float32)
        lh = jax.lax.dot_general(sel_h, dech, _TT,
                                 preferred_element_type=jnp.float32)
        theta = origin_ref[0, 4]
        c, s = jnp.cos(theta), jnp.sin(theta)
        ox = origin_ref[0, 0]
        oy = origin_ref[0, 1]
        npx = lpx * c - lpy * s + ox
        npy = lpx * s + lpy * c + oy
        t60 = jax.lax.broadcasted_iota(jnp.int32, (T_FUT, 1), 0)
        z1 = jnp.zeros((1, 1), jnp.float32)
        px_prev = jnp.concatenate([z1, npx[:-1, :]], axis=0)
        py_prev = jnp.concatenate([z1, npy[:-1, :]], axis=0)
        vx = jnp.where(t60 == 0, npx - ox, (npx - px_prev) * 10.0)
        vy = jnp.where(t60 == 0, npy - oy, (npy - py_prev) * 10.0)
        v49x = origin_ref[0, 2]
        v49y = origin_ref[0, 3]
        vx_prev = jnp.concatenate([z1, vx[:-1, :]], axis=0)
        vy_prev = jnp.concatenate([z1, vy[:-1, :]], axis=0)
        ax = jnp.where(t60 == 0, vx - v49x, (vx - vx_prev) * 10.0)
        ay = jnp.where(t60 == 0, vy - v49y, (vy - vy_prev) * 10.0)
        two_pi = 2.0 * jnp.pi
        hd = lh + theta
        hd = (hd + jnp.pi) % two_pi - jnp.pi
        act_s[...] = jnp.concatenate([hd, ax, ay], axis=1)
        acc_s[...] = jnp.zeros_like(acc_s)

    hd = act_s[:, 0:1]
    ax = act_s[:, 1:2]
    ay = act_s[:, 2:3]
    h1 = (hd * w1_ref[0:1, :] + ax * w1_ref[1:2, :] + ay * w1_ref[2:3, :]
          + b1_ref[...])
    h1 = jnp.maximum(h1, 0.0)
    acc_s[...] += jnp.dot(h1, w2_ref[...], preferred_element_type=jnp.float32)

    @pl.when(g == GRID - 1)
    def _finalize():
        h2 = acc_s[...] + b2_ref[...]
        y = jnp.dot(h2, w3_ref[...], preferred_element_type=jnp.float32) \
            + b3_ref[...]
        mean_ref[...] = jnp.tanh(y)
        std_ref[...] = jnp.log1p(jnp.exp(-jnp.abs(y))) + jnp.maximum(y, 0.0) \
            + 1e-8


def kernel(states, position, velocity, heading, predict_mask, agent_index,
           W_enc, W_dec_pos, W_dec_head, W1, b1, W2, b2, W3, b3):
    del predict_mask
    idx = jnp.asarray(agent_index, jnp.int32)
    pos_row = jax.lax.dynamic_slice_in_dim(position, idx, 1, axis=0)
    pos_row = pos_row.reshape(T_TOT, 3)
    vel49 = jax.lax.dynamic_slice(velocity, (idx, T_HIST - 1, 0),
                                  (1, 1, 3)).reshape(1, 3)
    th49 = jax.lax.dynamic_slice(heading, (idx, T_HIST - 1), (1, 1))
    origin = jnp.concatenate(
        [pos_row[T_HIST - 1:T_HIST, 0:2], vel49[:, 0:2], th49], axis=1)
    b1r = b1.reshape(1, HIDDEN)
    b2r = b2.reshape(1, HALF)
    b3r = b3.reshape(1, 3)

    mean, std = pl.pallas_call(
        _policy_kernel,
        grid=(GRID,),
        in_specs=[
            pl.BlockSpec((T_FUT, 4), lambda g: (0, 0)),
            pl.BlockSpec((T_TOT, 3), lambda g: (0, 0)),
            pl.BlockSpec((1, 5), lambda g: (0, 0)),
            pl.BlockSpec((TWO_T, D_ENC), lambda g: (0, 0)),
            pl.BlockSpec((D_ENC, MODES6 * T_FUT * 2), lambda g: (0, 0)),
            pl.BlockSpec((D_ENC, MODES6 * T_FUT), lambda g: (0, 0)),
            pl.BlockSpec((3, CHUNK), lambda g: (0, g)),
            pl.BlockSpec((1, CHUNK), lambda g: (0, g)),
            pl.BlockSpec((CHUNK, HALF), lambda g: (g, 0)),
            pl.BlockSpec((1, HALF), lambda g: (0, 0)),
            pl.BlockSpec((HALF, 3), lambda g: (0, 0)),
            pl.BlockSpec((1, 3), lambda g: (0, 0)),
        ],
        out_specs=[
            pl.BlockSpec((T_FUT, 3), lambda g: (0, 0)),
            pl.BlockSpec((T_FUT, 3), lambda g: (0, 0)),
        ],
        scratch_shapes=[
            pltpu.VMEM((T_FUT, 3), jnp.float32),
            pltpu.VMEM((T_FUT, HALF), jnp.float32),
        ],
        out_shape=[
            jax.ShapeDtypeStruct((T_FUT, 3), jnp.float32),
            jax.ShapeDtypeStruct((T_FUT, 3), jnp.float32),
        ],
    )(states, pos_row, origin,
      W_enc, W_dec_pos, W_dec_head,
      W1, b1r, W2, b2r, W3, b3r)
    return (mean, std)
